# Initial kernel scaffold; baseline (speedup 1.0000x reference)
import jax
import jax.numpy as jnp
from jax.experimental import pallas as pl


def kernel(x, edge_index, edge_weight, tc1_W1, tc1_b1, tc1_W2, tc1_b2, tc1_W3, tc1_b3, cheb_W, cheb_b, tc2_W1, tc2_b1, tc2_W2, tc2_b2, tc2_W3, tc2_b3, bn_g, bn_b, g1_W, g1_b, ln_g, ln_b, g2_W, g2_b, fc_W, fc_b):
    raise NotImplementedError("write your pallas kernel here")


# speedup vs baseline: 44.1035x; 44.1035x over previous
import functools

import jax
import jax.numpy as jnp
from jax import lax
from jax.experimental import pallas as pl
from jax.experimental.pallas import tpu as pltpu
from jax.experimental.pallas import tpu_sc as plsc

N = 10000
E = 160000
NC = 2
NS = 16
L = 16
NB = 1000
RPT = 624
CH = 16


def _mesh():
    return plsc.VectorSubcoreMesh(
        core_axis_name="c", subcore_axis_name="s", num_cores=NC, num_subcores=NS)


def _nq(s):
    return jnp.where(s == NS - 1, (N - RPT * (NS - 1)) // CH, RPT // CH)


def _sc_deg(src, dst, ew):
    K = 40
    ept = E // (NC * NS)
    nchunk = ept // K

    @functools.partial(
        pl.kernel,
        out_type=jax.ShapeDtypeStruct((NC, N, L), jnp.float32),
        mesh=_mesh(),
        compiler_params=pltpu.CompilerParams(needs_layout_passes=False, use_tc_tiling_on_sc=False),
        scratch_types=[
            pltpu.VMEM((K,), jnp.int32),
            pltpu.VMEM((K,), jnp.int32),
            pltpu.VMEM((48,), jnp.float32),
            pltpu.VMEM((K, L), jnp.float32),
            pltpu.VMEM((K, L), jnp.float32),
            pltpu.VMEM((CH, L), jnp.float32),
            pltpu.VMEM_SHARED((N, L), jnp.float32),
        ],
    )
    def k(src_hbm, dst_hbm, w_hbm, out_hbm, srcv, dstv, wv, rows1, rows2,
          stage, acc):
        c = lax.axis_index("c")
        s = lax.axis_index("s")
        z16 = jnp.zeros((L,), jnp.float32)
        lane = lax.broadcasted_iota(jnp.int32, (L,), 0)
        hot0 = jnp.where(lane == 0, 1.0, 0.0).astype(jnp.float32)
        hot1 = jnp.where(lane == 1, 1.0, 0.0).astype(jnp.float32)

        def init_rows(i, carry):
            rows1[i, pl.ds(0, L)] = z16
            rows2[i, pl.ds(0, L)] = hot1
            return carry
        lax.fori_loop(0, K, init_rows, 0)
        for i in range(CH):
            stage[i, pl.ds(0, L)] = z16

        def zchunk(q, carry):
            pltpu.sync_copy(stage, acc.at[pl.ds(s * RPT + q * CH, CH)])
            return carry
        lax.fori_loop(0, _nq(s), zchunk, 0)
        plsc.subcore_barrier()

        e0 = (c * NS + s) * ept

        def chunk(i, carry):
            base = e0 + i * K
            pltpu.sync_copy(src_hbm.at[pl.ds(base, K)], srcv)
            pltpu.sync_copy(dst_hbm.at[pl.ds(base, K)], dstv)
            pltpu.sync_copy(w_hbm.at[pl.ds(base, K)], wv.at[pl.ds(0, K)])
            for g in range(3):
                w16 = wv[pl.ds(g * L, L)]
                for e in range(L if g < 2 else K - 2 * L):
                    rows1[g * L + e, pl.ds(0, L)] = (
                        jnp.full((L,), w16[e], jnp.float32) * hot0)
            pltpu.sync_copy(rows1, acc.at[srcv], add=True)
            pltpu.sync_copy(rows2, acc.at[dstv], add=True)
            return carry
        lax.fori_loop(0, nchunk, chunk, 0)
        plsc.subcore_barrier()

        def wchunk(q, carry):
            r0 = s * RPT + q * CH
            pltpu.sync_copy(acc.at[pl.ds(r0, CH)], stage)
            pltpu.sync_copy(stage, out_hbm.at[c, pl.ds(r0, CH)])
            return carry
        lax.fori_loop(0, _nq(s), wchunk, 0)

    return k(src, dst, ew)


def _sc_edgeprep(src, dst, ew, dis1, dis2):
    K = 40
    ept = E // (NC * NS)
    nchunk = ept // K

    @functools.partial(
        pl.kernel,
        out_type=(jax.ShapeDtypeStruct((E,), jnp.float32),
                  jax.ShapeDtypeStruct((E,), jnp.float32)),
        mesh=_mesh(),
        compiler_params=pltpu.CompilerParams(needs_layout_passes=False, use_tc_tiling_on_sc=False),
        scratch_types=[
            pltpu.VMEM((48,), jnp.int32),
            pltpu.VMEM((48,), jnp.int32),
            pltpu.VMEM((48,), jnp.float32),
            pltpu.VMEM((48,), jnp.float32),
            pltpu.VMEM((48,), jnp.float32),
            pltpu.VMEM((N,), jnp.float32),
            pltpu.VMEM((N,), jnp.float32),
        ],
    )
    def k(src_hbm, dst_hbm, w_hbm, d1_hbm, d2_hbm, nw_hbm, cf_hbm,
          srcv, dstv, wv, nwv, cfv, d1v, d2v):
        c = lax.axis_index("c")
        s = lax.axis_index("s")
        pltpu.sync_copy(d1_hbm, d1v)
        pltpu.sync_copy(d2_hbm, d2v)
        lane = lax.broadcasted_iota(jnp.int32, (L,), 0)
        e0 = (c * NS + s) * ept

        def chunk(i, carry):
            base = e0 + i * K
            pltpu.sync_copy(src_hbm.at[pl.ds(base, K)], srcv.at[pl.ds(0, K)])
            pltpu.sync_copy(dst_hbm.at[pl.ds(base, K)], dstv.at[pl.ds(0, K)])
            pltpu.sync_copy(w_hbm.at[pl.ds(base, K)], wv.at[pl.ds(0, K)])
            for j in range(3):
                s16 = srcv[pl.ds(j * L, L)]
                d16 = dstv[pl.ds(j * L, L)]
                w16 = wv[pl.ds(j * L, L)]
                if j == 2:
                    m = lane < (K - 2 * L)
                    s16 = jnp.where(m, s16, 0)
                    d16 = jnp.where(m, d16, 0)
                g1s = plsc.load_gather(d1v, [s16])
                g1d = plsc.load_gather(d1v, [d16])
                nwv[pl.ds(j * L, L)] = -(g1s * w16 * g1d)
                g2s = plsc.load_gather(d2v, [s16])
                g2d = plsc.load_gather(d2v, [d16])
                cfv[pl.ds(j * L, L)] = g2s * g2d
            pltpu.sync_copy(nwv.at[pl.ds(0, K)], nw_hbm.at[pl.ds(base, K)])
            pltpu.sync_copy(cfv.at[pl.ds(0, K)], cf_hbm.at[pl.ds(base, K)])
            return carry
        lax.fori_loop(0, nchunk, chunk, 0)

    return k(src, dst, ew, dis1, dis2)


def _sc_matvec(x2, srcp, dst, w, ph):
    K = 80
    ept = E // NS
    nchunk = ept // K

    @functools.partial(
        pl.kernel,
        out_type=jax.ShapeDtypeStruct((NC, N, ph), jnp.float32),
        mesh=_mesh(),
        compiler_params=pltpu.CompilerParams(needs_layout_passes=False, use_tc_tiling_on_sc=False),
        scratch_types=[
            pltpu.VMEM((K,), jnp.int32),
            pltpu.VMEM((K,), jnp.int32),
            pltpu.VMEM((K,), jnp.float32),
            pltpu.VMEM((K, ph), jnp.float32),
            pltpu.VMEM((CH, ph), jnp.float32),
            pltpu.VMEM_SHARED((N, ph), jnp.float32),
            pltpu.SemaphoreType.DMA,
        ],
    )
    def k(x2_hbm, srcp_hbm, dst_hbm, w_hbm, out_hbm, srcv, dstv, wv, rows,
          stage, acc, sem):
        c = lax.axis_index("c")
        s = lax.axis_index("s")
        z16 = jnp.zeros((L,), jnp.float32)
        for i in range(CH):
            for j in range(ph // L):
                stage[i, pl.ds(j * L, L)] = z16

        def zchunk(q, carry):
            pltpu.sync_copy(stage, acc.at[pl.ds(s * RPT + q * CH, CH)])
            return carry
        lax.fori_loop(0, _nq(s), zchunk, 0)
        plsc.subcore_barrier()

        e0 = s * ept

        def chunk(i, carry):
            base = e0 + i * K
            pltpu.sync_copy(srcp_hbm.at[pl.ds(c * E + base, K)], srcv)
            pltpu.sync_copy(dst_hbm.at[pl.ds(base, K)], dstv)
            pltpu.sync_copy(w_hbm.at[pl.ds(base, K)], wv)
            pltpu.async_copy(x2_hbm.at[srcv], rows, sem).wait()

            def scale(g, carry2):
                w16 = wv[pl.ds(g * L, L)]
                for e in range(L):
                    wb = jnp.full((L,), w16[e], jnp.float32)
                    for j in range(ph // L):
                        rows[g * L + e, pl.ds(j * L, L)] = (
                            rows[g * L + e, pl.ds(j * L, L)] * wb)
                return carry2
            lax.fori_loop(0, K // L, scale, 0)
            pltpu.sync_copy(rows, acc.at[dstv], add=True)
            return carry
        lax.fori_loop(0, nchunk, chunk, 0)
        plsc.subcore_barrier()

        def wchunk(q, carry):
            r0 = s * RPT + q * CH
            pltpu.sync_copy(acc.at[pl.ds(r0, CH)], stage)
            pltpu.sync_copy(stage, out_hbm.at[c, pl.ds(r0, CH)])
            return carry
        lax.fori_loop(0, _nq(s), wchunk, 0)

    return k(x2, srcp, dst, w)


def _full(shape):
    return pl.BlockSpec(shape, lambda i: tuple(0 for _ in shape))


def _rows(width):
    return pl.BlockSpec((NB, width), lambda i: (i, 0))


def _split(ph):
    return pl.BlockSpec((NC, NB, ph), lambda i: (0, i, 0))


def _cat2(ref):
    return jnp.concatenate([ref[0], ref[1]], axis=1)


def _k1(xr, m1, m2, m3, b1, b2, b3):
    def body(x_ref, m1_ref, m2_ref, m3_ref, b1_ref, b2_ref, b3_ref, o_ref):
        xs = x_ref[...]
        dn = (((1,), (0,)), ((), ()))
        p = lax.dot_general(xs, m1_ref[...], dn) + b1_ref[...]
        q = lax.dot_general(xs, m2_ref[...], dn) + b2_ref[...]
        r = lax.dot_general(xs, m3_ref[...], dn) + b3_ref[...]
        h = jax.nn.relu(p * jax.nn.sigmoid(q) + r)
        o_ref[0] = h[:, :48]
        o_ref[1] = h[:, 48:]

    return pl.pallas_call(
        body, grid=(N // NB,),
        in_specs=[_rows(32), _full((32, 96)), _full((32, 96)), _full((32, 96)),
                  _full((1, 96)), _full((1, 96)), _full((1, 96))],
        out_specs=_split(48),
        out_shape=jax.ShapeDtypeStruct((NC, N, 48), jnp.float32),
    )(xr, m1, m2, m3, b1, b2, b3)


def _k2(d1p, d2p):
    def body(d1_ref, d2_ref, o1_ref, o2_ref, o3_ref):
        deg1 = jnp.sum(d1_ref[...], axis=0, keepdims=True)
        deg2 = jnp.sum(d2_ref[...], axis=0, keepdims=True) + 1.0
        o1_ref[...] = jnp.where(deg1 > 0, lax.rsqrt(deg1), 0.0)
        o2_ref[...] = lax.rsqrt(deg2)
        o3_ref[...] = 1.0 / deg2

    sds = jax.ShapeDtypeStruct((1, N), jnp.float32)
    return pl.pallas_call(
        body, grid=(1,),
        in_specs=[_full((NC, N)), _full((NC, N))],
        out_specs=[_full((1, N))] * 3,
        out_shape=[sds, sds, sds],
    )(d1p, d2p)


def _k3(x0s, t1s, t2s, bd0, bd1, bd2, cbt, m2a, m2b, m2c, b2a, b2b, b2c,
        bng, bnb, g1m):
    def body(x0_ref, t1_ref, t2_ref, bd0_ref, bd1_ref, bd2_ref, cbt_ref,
             m2a_ref, m2b_ref, m2c_ref, b2a_ref, b2b_ref, b2c_ref,
             bng_ref, bnb_ref, g1_ref, o_ref):
        dn = (((1,), (0,)), ((), ()))
        x0b = _cat2(x0_ref)
        t1b = _cat2(t1_ref)
        t2b = 2.0 * _cat2(t2_ref) - x0b
        ch = (lax.dot_general(x0b, bd0_ref[...], dn)
              + lax.dot_general(t1b, bd1_ref[...], dn)
              + lax.dot_general(t2b, bd2_ref[...], dn) + cbt_ref[...])
        ch = jax.nn.relu(ch)
        p = lax.dot_general(ch, m2a_ref[...], dn) + b2a_ref[...]
        q = lax.dot_general(ch, m2b_ref[...], dn) + b2b_ref[...]
        r = lax.dot_general(ch, m2c_ref[...], dn) + b2c_ref[...]
        h2 = jax.nn.relu(p * jax.nn.sigmoid(q) + r)
        mu = jnp.mean(h2, axis=1, keepdims=True)
        var = jnp.mean((h2 - mu) ** 2, axis=1, keepdims=True)
        hn = (h2 - mu) * lax.rsqrt(var + 1e-5)
        hn = hn * bng_ref[...] + bnb_ref[...]
        xw = lax.dot_general(hn, g1_ref[...], dn)
        o_ref[0] = xw[:, :128]
        o_ref[1] = xw[:, 128:]

    return pl.pallas_call(
        body, grid=(N // NB,),
        in_specs=[_split(48), _split(48), _split(48),
                  _full((96, 96)), _full((96, 96)), _full((96, 96)),
                  _full((1, 96)),
                  _full((96, 128)), _full((96, 128)), _full((96, 128)),
                  _full((1, 128)), _full((1, 128)), _full((1, 128)),
                  pl.BlockSpec((NB, 1), lambda i: (i, 0)),
                  pl.BlockSpec((NB, 1), lambda i: (i, 0)),
                  _full((128, 256))],
        out_specs=_split(128),
        out_shape=jax.ShapeDtypeStruct((NC, N, 128), jnp.float32),
    )(x0s, t1s, t2s, bd0, bd1, bd2, cbt, m2a, m2b, m2c, b2a, b2b, b2c,
      bng, bnb, g1m)


def _k4a(agg1, xw1, dd, g1bt):
    def body(a_ref, x_ref, dd_ref, b_ref, h_ref, s_ref, q_ref):
        h1 = _cat2(a_ref) + dd_ref[...] * _cat2(x_ref) + b_ref[...]
        h_ref[...] = h1
        z = jnp.zeros((7, 256), jnp.float32)
        s_ref[...] = jnp.concatenate(
            [jnp.sum(h1, axis=0, keepdims=True), z], axis=0)[None]
        q_ref[...] = jnp.concatenate(
            [jnp.sum(h1 * h1, axis=0, keepdims=True), z], axis=0)[None]

    nb = N // NB
    return pl.pallas_call(
        body, grid=(nb,),
        in_specs=[_split(128), _split(128), pl.BlockSpec((NB, 1), lambda i: (i, 0)),
                  _full((1, 256))],
        out_specs=[_rows(256), pl.BlockSpec((1, 8, 256), lambda i: (i, 0, 0)),
                   pl.BlockSpec((1, 8, 256), lambda i: (i, 0, 0))],
        out_shape=[jax.ShapeDtypeStruct((N, 256), jnp.float32),
                   jax.ShapeDtypeStruct((nb, 8, 256), jnp.float32),
                   jax.ShapeDtypeStruct((nb, 8, 256), jnp.float32)],
    )(agg1, xw1, dd, g1bt)


def _k4b(h1, ssum, ssq, lng, lnb, g2m, gmat, emat):
    cnt = float(N * 16)

    def body(h_ref, s_ref, q_ref, lg_ref, lb_ref, g2_ref, gm_ref, em_ref,
             o_ref):
        dn = (((1,), (0,)), ((), ()))
        tot = jnp.sum(jnp.sum(s_ref[...], axis=0), axis=0, keepdims=True)
        totq = jnp.sum(jnp.sum(q_ref[...], axis=0), axis=0, keepdims=True)
        mg = lax.dot_general(tot, gm_ref[...], dn) / cnt
        qg = lax.dot_general(totq, gm_ref[...], dn) / cnt
        vg = qg - mg * mg
        m256 = lax.dot_general(mg, em_ref[...], dn)
        r256 = lax.dot_general(lax.rsqrt(vg + 1e-5), em_ref[...], dn)
        lg = lg_ref[...]
        lb = lb_ref[...]
        lgt = jnp.concatenate([lg] * 16, axis=1)
        lbt = jnp.concatenate([lb] * 16, axis=1)
        hn = (h_ref[...] - m256) * r256 * lgt + lbt
        xw = lax.dot_general(hn, g2_ref[...], dn)
        o_ref[0] = xw[:, :64]
        o_ref[1] = xw[:, 64:]

    nb = N // NB
    return pl.pallas_call(
        body, grid=(nb,),
        in_specs=[_rows(256), _full((nb, 8, 256)), _full((nb, 8, 256)),
                  pl.BlockSpec((NB, 16), lambda i: (i, 0)),
                  pl.BlockSpec((NB, 16), lambda i: (i, 0)),
                  _full((256, 128)), _full((256, 16)), _full((16, 256))],
        out_specs=_split(64),
        out_shape=jax.ShapeDtypeStruct((NC, N, 64), jnp.float32),
    )(h1, ssum, ssq, lng, lnb, g2m, gmat, emat)


def _k5(agg2, xw2, dd, g2bt, fcm, fcbt):
    def body(a_ref, x_ref, dd_ref, b_ref, f_ref, fb_ref, o_ref):
        dn = (((1,), (0,)), ((), ()))
        h2 = _cat2(a_ref) + dd_ref[...] * _cat2(x_ref) + b_ref[...]
        o_ref[...] = lax.dot_general(h2, f_ref[...], dn) + fb_ref[...]

    return pl.pallas_call(
        body, grid=(N // NB,),
        in_specs=[_split(64), _split(64), pl.BlockSpec((NB, 1), lambda i: (i, 0)),
                  _full((1, 128)), _full((128, 16)), _full((1, 16))],
        out_specs=_rows(16),
        out_shape=jax.ShapeDtypeStruct((N, 16), jnp.float32),
    )(agg2, xw2, dd, g2bt, fcm, fcbt)


def _tconv_mat(w, nb_, tin, tout):
    k, cin, cout = w.shape
    m = jnp.zeros((nb_ * tin * cin, nb_ * tout * cout), jnp.float32)
    for b in range(nb_):
        for t in range(tout):
            for tau in range(k):
                m = m.at[b * tin * cin + (t + tau) * cin:
                         b * tin * cin + (t + tau + 1) * cin,
                         b * tout * cout + t * cout:
                         b * tout * cout + (t + 1) * cout].add(w[tau])
    return m


def _kron_eye(w, reps):
    return jnp.kron(jnp.eye(reps, dtype=jnp.float32), w)


def kernel(x, edge_index, edge_weight, tc1_W1, tc1_b1, tc1_W2, tc1_b2,
           tc1_W3, tc1_b3, cheb_W, cheb_b, tc2_W1, tc2_b1, tc2_W2, tc2_b2,
           tc2_W3, tc2_b3, bn_g, bn_b, g1_W, g1_b, ln_g, ln_b, g2_W, g2_b,
           fc_W, fc_b):
    src = edge_index[0]
    dst = edge_index[1]
    srcp = jnp.concatenate([src, N + src])

    xr = x.reshape(4, 8, N).transpose(2, 0, 1).reshape(N, 32)
    m1a = _tconv_mat(tc1_W1, 4, 8, 6)
    m1b = _tconv_mat(tc1_W2, 4, 8, 6)
    m1c = _tconv_mat(tc1_W3, 4, 8, 6)
    b1a = jnp.tile(tc1_b1, 24)[None]
    b1b = jnp.tile(tc1_b2, 24)[None]
    b1c = jnp.tile(tc1_b3, 24)[None]
    bd0 = _kron_eye(cheb_W[0], 24)
    bd1 = _kron_eye(cheb_W[1], 24)
    bd2 = _kron_eye(cheb_W[2], 24)
    cbt = jnp.tile(cheb_b, 24)[None]
    m2a = _tconv_mat(tc2_W1, 4, 6, 4)
    m2b = _tconv_mat(tc2_W2, 4, 6, 4)
    m2c = _tconv_mat(tc2_W3, 4, 6, 4)
    b2a = jnp.tile(tc2_b1, 16)[None]
    b2b = jnp.tile(tc2_b2, 16)[None]
    b2c = jnp.tile(tc2_b3, 16)[None]
    g1m = _kron_eye(g1_W, 16)
    g1bt = jnp.tile(g1_b, 16)[None]
    g2m = _kron_eye(g2_W, 16)
    g2bt = jnp.tile(g2_b, 16)[None]
    fcm = _kron_eye(fc_W, 16)
    fcbt = jnp.tile(fc_b, 16)[None]
    gmat = _kron_eye(jnp.ones((16, 1), jnp.float32), 16)
    emat = _kron_eye(jnp.ones((1, 16), jnp.float32), 16)

    x0s = _k1(xr, m1a, m1b, m1c, b1a, b1b, b1c)
    degp = _sc_deg(src, dst, edge_weight)
    dis1, dis2, dd = _k2(degp[:, :, 0], degp[:, :, 1])
    nw, coef = _sc_edgeprep(src, dst, edge_weight,
                            dis1.reshape(N), dis2.reshape(N))
    dd_col = dd.reshape(N, 1)

    t1s = _sc_matvec(x0s.reshape(NC * N, 48), srcp, dst, nw, 48)
    t2s = _sc_matvec(t1s.reshape(NC * N, 48), srcp, dst, nw, 48)
    xw1 = _k3(x0s, t1s, t2s, bd0, bd1, bd2, cbt, m2a, m2b, m2c,
              b2a, b2b, b2c, bn_g.reshape(N, 1), bn_b.reshape(N, 1), g1m)

    agg1 = _sc_matvec(xw1.reshape(NC * N, 128), srcp, dst, coef, 128)
    h1, ssum, ssq = _k4a(agg1, xw1, dd_col, g1bt)
    xw2 = _k4b(h1, ssum, ssq, ln_g, ln_b, g2m, gmat, emat)

    agg2 = _sc_matvec(xw2.reshape(NC * N, 64), srcp, dst, coef, 64)
    o = _k5(agg2, xw2, dd_col, g2bt, fcm, fcbt)
    return o.reshape(N, 4, 4).transpose(1, 2, 0)[:, :, :, None]
